# baseline (speedup 1.0000x reference)
import functools

import numpy as np
import jax
import jax.numpy as jnp
from jax import lax
from jax.experimental import pallas as pl
from jax.experimental.pallas import tpu as pltpu
from jax.experimental.pallas import tpu_sc as plsc

_NUM_LAYER = 3
_EMB = 512
_N_NODE = 10000
_N_NET = 10000
_E = 160000
_DROP_P = 0.4
_NCHUNK = 4
_CW = 128
_B = 64
_NW = 32
_ACC_ROWS = 10112
_RPT = _ACC_ROWS // 16
_BM = 1000

_U32 = np.uint64(0xFFFFFFFF)


def _threefry2x32(k0, k1, x0, x1):
    x0 = np.asarray(x0, np.uint64)
    x1 = np.asarray(x1, np.uint64)
    ks = [np.uint64(k0), np.uint64(k1),
          np.uint64(k0) ^ np.uint64(k1) ^ np.uint64(0x1BD11BDA)]
    rot = ((13, 15, 26, 6), (17, 29, 16, 24))
    x0 = (x0 + ks[0]) & _U32
    x1 = (x1 + ks[1]) & _U32
    for i in range(5):
        for r in rot[i % 2]:
            x0 = (x0 + x1) & _U32
            r64 = np.uint64(r)
            x1 = ((x1 << r64 | x1 >> (np.uint64(32) - r64)) & _U32) ^ x0
        x0 = (x0 + ks[(i + 1) % 3]) & _U32
        x1 = (x1 + ks[(i + 2) % 3] + np.uint64(i + 1)) & _U32
    return x0.astype(np.uint32), x1.astype(np.uint32)


def _uniform01(seed, n):
    idx = np.arange(n, dtype=np.uint64)
    hi = (idx >> np.uint64(32)).astype(np.uint32)
    lo = (idx & _U32).astype(np.uint32)
    a, b = _threefry2x32(0, seed, hi, lo)
    bits = a ^ b
    u = ((bits >> np.uint32(9)) | np.uint32(0x3F800000)).view(np.float32)
    return np.maximum(np.float32(0.0), u - np.float32(1.0))


_MASK = _uniform01(42, _E) >= _DROP_P
_KEEP = np.nonzero(_MASK)[0].astype(np.int32)
_KN = int(_KEEP.shape[0])


def _round_up(n, m):
    return (n + m - 1) // m * m


_UPAD = _round_up(_E, 2 * _B * _NW)
_WPAD = _round_up(_KN, 2 * _B * _NW)


def _lrelu(x):
    return jnp.where(x >= 0, x, 0.1 * x)


def _lin_body(x_ref, w_ref, b_ref, o_ref):
    y = jnp.dot(x_ref[...], w_ref[...], preferred_element_type=jnp.float32)
    o_ref[...] = _lrelu(y + b_ref[...])


def _linear_lrelu(x, w, b):
    m, k = x.shape
    n = w.shape[1]
    return pl.pallas_call(
        _lin_body,
        grid=(m // _BM,),
        in_specs=[
            pl.BlockSpec((_BM, k), lambda i: (i, 0)),
            pl.BlockSpec((k, n), lambda i: (0, 0)),
            pl.BlockSpec((1, n), lambda i: (0, 0)),
        ],
        out_specs=pl.BlockSpec((_BM, n), lambda i: (i, 0)),
        out_shape=jax.ShapeDtypeStruct((m, n), jnp.float32),
    )(x, w, b.reshape(1, n))


def _msg_body(x_ref, w_ref, b_ref, o_ref):
    y = _lrelu(jnp.dot(x_ref[...], w_ref[...], preferred_element_type=jnp.float32)
               + b_ref[...])
    for c in range(_NCHUNK):
        o_ref[c] = y[:, c * _CW:(c + 1) * _CW]


def _msg_mm(x, w, b):
    m, k = x.shape
    n = w.shape[1]
    return pl.pallas_call(
        _msg_body,
        grid=(m // _BM,),
        in_specs=[
            pl.BlockSpec((_BM, k), lambda i: (i, 0)),
            pl.BlockSpec((k, n), lambda i: (0, 0)),
            pl.BlockSpec((1, n), lambda i: (0, 0)),
        ],
        out_specs=pl.BlockSpec((_NCHUNK, _BM, _CW), lambda i: (0, i, 0)),
        out_shape=jax.ShapeDtypeStruct((_NCHUNK, m, _CW), jnp.float32),
    )(x, w, b.reshape(1, n))


def _hn_body(hn_ref, agg_ref, wt_ref, wb_ref, b_ref, opre_ref, ores_ref):
    hn = hn_ref[...]
    acc = jnp.dot(hn, wt_ref[...], preferred_element_type=jnp.float32)
    wb = wb_ref[...]
    for core in range(2):
        for c in range(_NCHUNK):
            acc += jnp.dot(agg_ref[core, c], wb[c * _CW:(c + 1) * _CW],
                           preferred_element_type=jnp.float32)
    pre = _lrelu(acc + b_ref[...])
    for c in range(_NCHUNK):
        opre_ref[c] = pre[:, c * _CW:(c + 1) * _CW]
    ores_ref[...] = pre + hn


def _hn_mm(hn, agg, wt, wb, b):
    m, n = hn.shape
    return pl.pallas_call(
        _hn_body,
        grid=(m // _BM,),
        in_specs=[
            pl.BlockSpec((_BM, n), lambda i: (i, 0)),
            pl.BlockSpec((2, _NCHUNK, _BM, _CW), lambda i: (0, 0, i, 0)),
            pl.BlockSpec((n, n), lambda i: (0, 0)),
            pl.BlockSpec((n, n), lambda i: (0, 0)),
            pl.BlockSpec((1, n), lambda i: (0, 0)),
        ],
        out_specs=[
            pl.BlockSpec((_NCHUNK, _BM, _CW), lambda i: (0, i, 0)),
            pl.BlockSpec((_BM, n), lambda i: (i, 0)),
        ],
        out_shape=[
            jax.ShapeDtypeStruct((_NCHUNK, m, _CW), jnp.float32),
            jax.ShapeDtypeStruct((m, n), jnp.float32),
        ],
    )(hn, agg, wt, wb, b.reshape(1, n))


def _h_body(h_ref, agg_ref, wt_ref, wb_ref, b_ref, o_ref):
    h = h_ref[...]
    acc = jnp.dot(h, wt_ref[...], preferred_element_type=jnp.float32)
    wb = wb_ref[...]
    for core in range(2):
        for c in range(_NCHUNK):
            acc += jnp.dot(agg_ref[core, c], wb[c * _CW:(c + 1) * _CW],
                           preferred_element_type=jnp.float32)
    o_ref[...] = _lrelu(acc + b_ref[...]) + h


def _h_mm(h, agg, wt, wb, b):
    m, n = h.shape
    return pl.pallas_call(
        _h_body,
        grid=(m // _BM,),
        in_specs=[
            pl.BlockSpec((_BM, n), lambda i: (i, 0)),
            pl.BlockSpec((2, _NCHUNK, _BM, _CW), lambda i: (0, 0, i, 0)),
            pl.BlockSpec((n, n), lambda i: (0, 0)),
            pl.BlockSpec((n, n), lambda i: (0, 0)),
            pl.BlockSpec((1, n), lambda i: (0, 0)),
        ],
        out_specs=pl.BlockSpec((_BM, n), lambda i: (i, 0)),
        out_shape=jax.ShapeDtypeStruct((m, n), jnp.float32),
    )(h, agg, wt, wb, b.reshape(1, n))


def _head_node_body(h0, h1, h2, h3, w1, b1, w2, b2, wf, bf, o_ref):
    hs = (h0, h1, h2, h3)
    acc = b1[...].astype(jnp.float32) * jnp.ones((_BM, 1), jnp.float32)
    for i in range(4):
        acc += jnp.dot(hs[i][...], w1[i], preferred_element_type=jnp.float32)
    t = _lrelu(acc)
    t = _lrelu(jnp.dot(t, w2[...], preferred_element_type=jnp.float32) + b2[...])
    o_ref[...] = jnp.dot(t, wf[...], preferred_element_type=jnp.float32) + bf[...]


def _head_node(h_list, w1, b1, w2, b2, wf, bf):
    m, n = h_list[0].shape
    w1r = w1.reshape(4, n, 256)
    return pl.pallas_call(
        _head_node_body,
        grid=(m // _BM,),
        in_specs=[pl.BlockSpec((_BM, n), lambda i: (i, 0)) for _ in range(4)] + [
            pl.BlockSpec((4, n, 256), lambda i: (0, 0, 0)),
            pl.BlockSpec((1, 256), lambda i: (0, 0)),
            pl.BlockSpec((256, 256), lambda i: (0, 0)),
            pl.BlockSpec((1, 256), lambda i: (0, 0)),
            pl.BlockSpec((256, 1), lambda i: (0, 0)),
            pl.BlockSpec((1, 1), lambda i: (0, 0)),
        ],
        out_specs=pl.BlockSpec((_BM, 1), lambda i: (i, 0)),
        out_shape=jax.ShapeDtypeStruct((m, 1), jnp.float32),
    )(*h_list, w1r, b1.reshape(1, 256), w2, b2.reshape(1, 256), wf,
      bf.reshape(1, 1))


def _head_net_body(h0, h1, h2, h3, w1, b1, w2, b2, o_ref):
    hs = (h0, h1, h2, h3)
    acc = b1[...].astype(jnp.float32) * jnp.ones((_BM, 1), jnp.float32)
    for i in range(4):
        acc += jnp.dot(hs[i][...], w1[i], preferred_element_type=jnp.float32)
    t = _lrelu(acc)
    o_ref[...] = jnp.abs(
        _lrelu(jnp.dot(t, w2[...], preferred_element_type=jnp.float32) + b2[...]))


def _head_net(h_list, w1, b1, w2, b2):
    m, n = h_list[0].shape
    w1r = w1.reshape(4, n, 64)
    return pl.pallas_call(
        _head_net_body,
        grid=(m // _BM,),
        in_specs=[pl.BlockSpec((_BM, n), lambda i: (i, 0)) for _ in range(4)] + [
            pl.BlockSpec((4, n, 64), lambda i: (0, 0, 0)),
            pl.BlockSpec((1, 64), lambda i: (0, 0)),
            pl.BlockSpec((64, 64), lambda i: (0, 0)),
            pl.BlockSpec((1, 64), lambda i: (0, 0)),
        ],
        out_specs=pl.BlockSpec((_BM, 64), lambda i: (i, 0)),
        out_shape=jax.ShapeDtypeStruct((m, 64), jnp.float32),
    )(*h_list, w1r, b1.reshape(1, 64), w2, b2.reshape(1, 64))


_NBUF = 2
_NBU = _UPAD // (_NW * _B)
_NBW = _WPAD // (_NW * _B)


@functools.lru_cache(maxsize=None)
def _sc_agg():
    mesh = plsc.VectorSubcoreMesh(core_axis_name="c", subcore_axis_name="s",
                                  num_cores=2, num_subcores=16)

    def body(cnt_ref, gu_ref, du_ref, gw_ref, dw_ref, w_ref, tab_ref, z_ref,
             out_ref, rows_v, cnt_v, gi_v, di_v, w_v, accum,
             gsem0, gsem1, ssem0, ssem1):
        gsems = (gsem0, gsem1)
        ssems = (ssem0, ssem1)
        c = lax.axis_index("c")
        s = lax.axis_index("s")
        wid = c * 16 + s

        pltpu.sync_copy(cnt_ref, cnt_v)
        lanes = lax.iota(jnp.int32, 16)
        cnt = cnt_v[...]
        nbu = jnp.sum(jnp.where(lanes == 0, cnt, 0))
        nbw = jnp.sum(jnp.where(lanes == 1, cnt, 0))

        pltpu.sync_copy(w_ref.at[wid], w_v)

        def bump(g_v, nrow, off):
            def rbody(r, cc):
                for kk in range(_B // 16):
                    g_v[r, pl.ds(kk * 16, 16)] = (
                        g_v[r, pl.ds(kk * 16, 16)] + off)
                return cc

            lax.fori_loop(0, nrow, rbody, 0)

        def run_phase(g_v, d_v, nb, weighted):
            def start_g(j, b):
                pltpu.async_copy(tab_ref.at[g_v.at[j]], rows_v.at[b],
                                 gsems[b])

            def start_s(j, b):
                pltpu.async_copy(rows_v.at[b], accum.at[d_v.at[j]],
                                 ssems[b], add=True)

            def drain(sem, b):
                pltpu.make_async_copy(tab_ref.at[g_v.at[0]], rows_v.at[b],
                                      sem).wait()

            def mult(j, b):
                def ebody(e, cc):
                    jv = jnp.zeros((16,), jnp.int32) + j
                    ev = jnp.zeros((16,), jnp.int32) + e
                    wb = plsc.load_gather(w_v, [jv, ev])
                    for kk in range(_CW // 16):
                        rows_v[b, e, pl.ds(kk * 16, 16)] = (
                            rows_v[b, e, pl.ds(kk * 16, 16)] * wb)
                    return cc

                lax.fori_loop(0, _B, ebody, 0)

            @pl.when(nb > 0)
            def _():
                start_g(0, 0)

            def pair(q, carry):
                j0 = 2 * q
                drain(gsems[0], 0)
                if weighted:
                    mult(j0, 0)

                @pl.when(q >= 1)
                def _():
                    drain(ssems[1], 1)

                start_g(j0 + 1, 1)
                start_s(j0, 0)
                drain(gsems[1], 1)
                if weighted:
                    mult(j0 + 1, 1)

                @pl.when(j0 + 2 < nb)
                def _():
                    drain(ssems[0], 0)
                    start_g(j0 + 2, 0)

                start_s(j0 + 1, 1)
                return carry

            lax.fori_loop(0, nb // 2, pair, 0)

            @pl.when(nb > 0)
            def _():
                drain(ssems[0], 0)
                drain(ssems[1], 1)

        for cf in range(_NCHUNK):
            pltpu.sync_copy(z_ref, accum.at[pl.ds(s * _RPT, _RPT)])
            pltpu.sync_copy(gu_ref.at[wid], gi_v)
            pltpu.sync_copy(du_ref.at[wid], di_v)
            if cf:
                bump(gi_v, _NBU, cf * _N_NODE)
            plsc.subcore_barrier()
            run_phase(gi_v, di_v, nbu, False)
            pltpu.sync_copy(gw_ref.at[wid], gi_v.at[pl.ds(0, _NBW)])
            pltpu.sync_copy(dw_ref.at[wid], di_v.at[pl.ds(0, _NBW)])
            if cf:
                bump(gi_v, _NBW, cf * _N_NODE)
            run_phase(gi_v, di_v, nbw, True)
            plsc.subcore_barrier()
            pltpu.sync_copy(accum.at[pl.ds(s * _RPT, _RPT)],
                            out_ref.at[c, cf].at[pl.ds(s * _RPT, _RPT)])
            plsc.subcore_barrier()

    scratch = [
        pltpu.VMEM((_NBUF, _B, _CW), jnp.float32),
        pltpu.VMEM((16,), jnp.int32),
        pltpu.VMEM((_NBU, _B), jnp.int32),
        pltpu.VMEM((_NBU, _B), jnp.int32),
        pltpu.VMEM((_NBW, _B), jnp.float32),
        pltpu.VMEM_SHARED((_ACC_ROWS, _CW), jnp.float32),
    ] + [pltpu.SemaphoreType.DMA] * 4

    return pl.kernel(
        body,
        out_type=jax.ShapeDtypeStruct((2, _NCHUNK, _ACC_ROWS, _CW),
                                      jnp.float32),
        mesh=mesh,
        compiler_params=pltpu.CompilerParams(needs_layout_passes=False),
        scratch_types=scratch,
    )


def _pad1(x, n, val):
    return jnp.concatenate(
        [x, jnp.full((n - x.shape[0],), val, x.dtype)]).reshape(_NW, -1, _B)


def kernel(node_x, net_x, edge_weight_sink_to_net, params,
           edge_index_sink_to_net, edge_index_source_to_net):
    p = params
    ew = edge_weight_sink_to_net

    h = _linear_lrelu(node_x, p['enc_W1'], p['enc_b1'])
    h = _linear_lrelu(h, p['enc_W2'], p['enc_b2'])
    hn = _linear_lrelu(net_x, p['net_W'], p['net_b'])

    src_nodes = edge_index_source_to_net[0].astype(jnp.int32)
    src_nets = edge_index_source_to_net[1].astype(jnp.int32)
    sink_nodes = edge_index_sink_to_net[0][_KEEP].astype(jnp.int32)
    sink_nets = edge_index_sink_to_net[1][_KEEP].astype(jnp.int32)
    kw = ew[_KEEP]

    po = jnp.argsort(src_nets)
    src_nets = src_nets[po]
    src_nodes = src_nodes[po]
    pw = jnp.argsort(sink_nets)
    w_nets = sink_nets[pw]
    w_nodes = sink_nodes[pw]
    w_w = kw[pw]
    pn = jnp.argsort(sink_nodes)
    n_nodes = sink_nodes[pn]
    n_nets = sink_nets[pn]

    gu = _pad1(src_nodes, _UPAD, 0)
    du = _pad1(src_nets, _UPAD, _N_NET)
    gw = _pad1(w_nodes, _WPAD, 0)
    dw = _pad1(w_nets, _WPAD, _N_NET)
    wv = _pad1(w_w, _WPAD, 0.0)
    gn = jnp.pad(_pad1(n_nets, _WPAD, 0),
                 ((0, 0), (0, _NBU - _NBW), (0, 0)))
    dn = jnp.pad(_pad1(n_nodes, _WPAD, _N_NET),
                 ((0, 0), (0, _NBU - _NBW), (0, 0)),
                 constant_values=_N_NET)
    zeros = jnp.zeros((_RPT, _CW), jnp.float32)
    gw_dummy = jnp.zeros((_NW, _NBW, _B), jnp.int32)
    dw_dummy = jnp.full((_NW, _NBW, _B), _N_NET, jnp.int32)
    w_dummy = jnp.zeros((_NW, _NBW, _B), jnp.float32)
    cnt_net = jnp.asarray(np.array([_NBU, _NBW] + [0] * 14, np.int32))
    cnt_node = jnp.asarray(np.array([_NBW, 0] + [0] * 14, np.int32))

    agg_fn = _sc_agg()

    ls = p['layers']
    is_net = jnp.asarray(np.array([True, False] * _NUM_LAYER))
    xs = {
        'is_net': is_net,
        'gu': jnp.stack([gu, gn] * _NUM_LAYER),
        'du': jnp.stack([du, dn] * _NUM_LAYER),
        'gw': jnp.stack([gw, gw_dummy] * _NUM_LAYER),
        'dw': jnp.stack([dw, dw_dummy] * _NUM_LAYER),
        'wv': jnp.stack([wv, w_dummy] * _NUM_LAYER),
        'cnt': jnp.stack([cnt_net, cnt_node] * _NUM_LAYER),
        'wm': jnp.stack([ls[l]['W_msg'] for l in range(_NUM_LAYER)
                         for _ in range(2)]),
        'bm': jnp.stack([ls[l]['b_msg'] for l in range(_NUM_LAYER)
                         for _ in range(2)]),
        'wt': jnp.stack([w for l in range(_NUM_LAYER)
                         for w in (ls[l]['W_net'][:_EMB],
                                   ls[l]['W_node'][:_EMB])]),
        'wb': jnp.stack([w for l in range(_NUM_LAYER)
                         for w in (ls[l]['W_net'][_EMB:],
                                   ls[l]['W_node'][_EMB:])]),
        'bb': jnp.stack([w for l in range(_NUM_LAYER)
                         for w in (ls[l]['b_net'], ls[l]['b_node'])]),
    }

    def step(carry, x):
        hc, hnc, tbl = carry
        tbl = lax.cond(x['is_net'],
                       lambda: _msg_mm(hc, x['wm'], x['bm']),
                       lambda: tbl)
        agg = agg_fn(x['cnt'], x['gu'], x['du'], x['gw'], x['dw'], x['wv'],
                     tbl.reshape(_NCHUNK * _N_NODE, _CW), zeros)

        def net_branch():
            pre_cm, hn_new = _hn_mm(hnc, agg, x['wt'], x['wb'], x['bb'])
            return hc, hn_new, pre_cm

        def node_branch():
            h_new = _h_mm(hc, agg, x['wt'], x['wb'], x['bb'])
            return h_new, hnc, tbl

        carry2 = lax.cond(x['is_net'], net_branch, node_branch)
        return carry2, (carry2[0], carry2[1])

    tbl0 = jnp.zeros((_NCHUNK, _N_NODE, _CW), jnp.float32)
    _, (hs, hns) = lax.scan(step, (h, hn, tbl0), xs)

    h_list = [h, hs[1], hs[3], hs[5]]
    hn_list = [hn, hns[0], hns[2], hns[4]]

    node_out = _head_node(h_list, p['fc1n_W'], p['fc1n_b'],
                          p['fc2n_W'], p['fc2n_b'], p['final_W'], p['final_b'])
    net_out = _head_net(hn_list, p['fc1e_W'], p['fc1e_b'],
                        p['fc2e_W'], p['fc2e_b'])
    return node_out, net_out


# speedup vs baseline: 1.3927x; 1.3927x over previous
import functools

import numpy as np
import jax
import jax.numpy as jnp
from jax import lax
from jax.experimental import pallas as pl
from jax.experimental.pallas import tpu as pltpu
from jax.experimental.pallas import tpu_sc as plsc

_NUM_LAYER = 3
_EMB = 512
_N_NODE = 10000
_N_NET = 10000
_E = 160000
_DROP_P = 0.4
_NCHUNK = 4
_CW = 128
_B = 64
_NW = 32
_ACC_ROWS = 10112
_RPT = _ACC_ROWS // 16
_BM = 1000

_U32 = np.uint64(0xFFFFFFFF)


def _threefry2x32(k0, k1, x0, x1):
    x0 = np.asarray(x0, np.uint64)
    x1 = np.asarray(x1, np.uint64)
    ks = [np.uint64(k0), np.uint64(k1),
          np.uint64(k0) ^ np.uint64(k1) ^ np.uint64(0x1BD11BDA)]
    rot = ((13, 15, 26, 6), (17, 29, 16, 24))
    x0 = (x0 + ks[0]) & _U32
    x1 = (x1 + ks[1]) & _U32
    for i in range(5):
        for r in rot[i % 2]:
            x0 = (x0 + x1) & _U32
            r64 = np.uint64(r)
            x1 = ((x1 << r64 | x1 >> (np.uint64(32) - r64)) & _U32) ^ x0
        x0 = (x0 + ks[(i + 1) % 3]) & _U32
        x1 = (x1 + ks[(i + 2) % 3] + np.uint64(i + 1)) & _U32
    return x0.astype(np.uint32), x1.astype(np.uint32)


def _uniform01(seed, n):
    idx = np.arange(n, dtype=np.uint64)
    hi = (idx >> np.uint64(32)).astype(np.uint32)
    lo = (idx & _U32).astype(np.uint32)
    a, b = _threefry2x32(0, seed, hi, lo)
    bits = a ^ b
    u = ((bits >> np.uint32(9)) | np.uint32(0x3F800000)).view(np.float32)
    return np.maximum(np.float32(0.0), u - np.float32(1.0))


_MASK = _uniform01(42, _E) >= _DROP_P
_KEEP = np.nonzero(_MASK)[0].astype(np.int32)
_KN = int(_KEEP.shape[0])


def _round_up(n, m):
    return (n + m - 1) // m * m


_UPAD = _round_up(_E, 2 * _B * _NW)
_WPAD = _round_up(_KN, 2 * _B * _NW)


def _lrelu(x):
    return jnp.where(x >= 0, x, 0.1 * x)


def _lin_body(x_ref, w_ref, b_ref, o_ref):
    y = jnp.dot(x_ref[...], w_ref[...], preferred_element_type=jnp.float32)
    o_ref[...] = _lrelu(y + b_ref[...])


def _linear_lrelu(x, w, b):
    m, k = x.shape
    n = w.shape[1]
    return pl.pallas_call(
        _lin_body,
        grid=(m // _BM,),
        in_specs=[
            pl.BlockSpec((_BM, k), lambda i: (i, 0)),
            pl.BlockSpec((k, n), lambda i: (0, 0)),
            pl.BlockSpec((1, n), lambda i: (0, 0)),
        ],
        out_specs=pl.BlockSpec((_BM, n), lambda i: (i, 0)),
        out_shape=jax.ShapeDtypeStruct((m, n), jnp.float32),
    )(x, w, b.reshape(1, n))


def _msg_body(x_ref, w_ref, b_ref, o_ref):
    y = _lrelu(jnp.dot(x_ref[...], w_ref[...], preferred_element_type=jnp.float32)
               + b_ref[...])
    for c in range(_NCHUNK):
        o_ref[c] = y[:, c * _CW:(c + 1) * _CW]


def _msg_mm(x, w, b):
    m, k = x.shape
    n = w.shape[1]
    return pl.pallas_call(
        _msg_body,
        grid=(m // _BM,),
        in_specs=[
            pl.BlockSpec((_BM, k), lambda i: (i, 0)),
            pl.BlockSpec((k, n), lambda i: (0, 0)),
            pl.BlockSpec((1, n), lambda i: (0, 0)),
        ],
        out_specs=pl.BlockSpec((_NCHUNK, _BM, _CW), lambda i: (0, i, 0)),
        out_shape=jax.ShapeDtypeStruct((_NCHUNK, m, _CW), jnp.float32),
    )(x, w, b.reshape(1, n))


def _hn_body(hn_ref, agg_ref, wt_ref, wb_ref, b_ref, opre_ref, ores_ref):
    hn = hn_ref[...]
    acc = jnp.dot(hn, wt_ref[...], preferred_element_type=jnp.float32)
    wb = wb_ref[...]
    for core in range(2):
        for c in range(_NCHUNK):
            acc += jnp.dot(agg_ref[core, c], wb[c * _CW:(c + 1) * _CW],
                           preferred_element_type=jnp.float32)
    pre = _lrelu(acc + b_ref[...])
    for c in range(_NCHUNK):
        opre_ref[c] = pre[:, c * _CW:(c + 1) * _CW]
    ores_ref[...] = pre + hn


def _hn_mm(hn, agg, wt, wb, b):
    m, n = hn.shape
    return pl.pallas_call(
        _hn_body,
        grid=(m // _BM,),
        in_specs=[
            pl.BlockSpec((_BM, n), lambda i: (i, 0)),
            pl.BlockSpec((2, _NCHUNK, _BM, _CW), lambda i: (0, 0, i, 0)),
            pl.BlockSpec((n, n), lambda i: (0, 0)),
            pl.BlockSpec((n, n), lambda i: (0, 0)),
            pl.BlockSpec((1, n), lambda i: (0, 0)),
        ],
        out_specs=[
            pl.BlockSpec((_NCHUNK, _BM, _CW), lambda i: (0, i, 0)),
            pl.BlockSpec((_BM, n), lambda i: (i, 0)),
        ],
        out_shape=[
            jax.ShapeDtypeStruct((_NCHUNK, m, _CW), jnp.float32),
            jax.ShapeDtypeStruct((m, n), jnp.float32),
        ],
    )(hn, agg, wt, wb, b.reshape(1, n))


def _h_body(h_ref, agg_ref, wt_ref, wb_ref, b_ref, o_ref):
    h = h_ref[...]
    acc = jnp.dot(h, wt_ref[...], preferred_element_type=jnp.float32)
    wb = wb_ref[...]
    for core in range(2):
        for c in range(_NCHUNK):
            acc += jnp.dot(agg_ref[core, c], wb[c * _CW:(c + 1) * _CW],
                           preferred_element_type=jnp.float32)
    o_ref[...] = _lrelu(acc + b_ref[...]) + h


def _h_mm(h, agg, wt, wb, b):
    m, n = h.shape
    return pl.pallas_call(
        _h_body,
        grid=(m // _BM,),
        in_specs=[
            pl.BlockSpec((_BM, n), lambda i: (i, 0)),
            pl.BlockSpec((2, _NCHUNK, _BM, _CW), lambda i: (0, 0, i, 0)),
            pl.BlockSpec((n, n), lambda i: (0, 0)),
            pl.BlockSpec((n, n), lambda i: (0, 0)),
            pl.BlockSpec((1, n), lambda i: (0, 0)),
        ],
        out_specs=pl.BlockSpec((_BM, n), lambda i: (i, 0)),
        out_shape=jax.ShapeDtypeStruct((m, n), jnp.float32),
    )(h, agg, wt, wb, b.reshape(1, n))


def _head_node_body(h0, h1, h2, h3, w1, b1, w2, b2, wf, bf, o_ref):
    hs = (h0, h1, h2, h3)
    acc = b1[...].astype(jnp.float32) * jnp.ones((_BM, 1), jnp.float32)
    for i in range(4):
        acc += jnp.dot(hs[i][...], w1[i], preferred_element_type=jnp.float32)
    t = _lrelu(acc)
    t = _lrelu(jnp.dot(t, w2[...], preferred_element_type=jnp.float32) + b2[...])
    o_ref[...] = jnp.dot(t, wf[...], preferred_element_type=jnp.float32) + bf[...]


def _head_node(h_list, w1, b1, w2, b2, wf, bf):
    m, n = h_list[0].shape
    w1r = w1.reshape(4, n, 256)
    return pl.pallas_call(
        _head_node_body,
        grid=(m // _BM,),
        in_specs=[pl.BlockSpec((_BM, n), lambda i: (i, 0)) for _ in range(4)] + [
            pl.BlockSpec((4, n, 256), lambda i: (0, 0, 0)),
            pl.BlockSpec((1, 256), lambda i: (0, 0)),
            pl.BlockSpec((256, 256), lambda i: (0, 0)),
            pl.BlockSpec((1, 256), lambda i: (0, 0)),
            pl.BlockSpec((256, 1), lambda i: (0, 0)),
            pl.BlockSpec((1, 1), lambda i: (0, 0)),
        ],
        out_specs=pl.BlockSpec((_BM, 1), lambda i: (i, 0)),
        out_shape=jax.ShapeDtypeStruct((m, 1), jnp.float32),
    )(*h_list, w1r, b1.reshape(1, 256), w2, b2.reshape(1, 256), wf,
      bf.reshape(1, 1))


def _head_net_body(h0, h1, h2, h3, w1, b1, w2, b2, o_ref):
    hs = (h0, h1, h2, h3)
    acc = b1[...].astype(jnp.float32) * jnp.ones((_BM, 1), jnp.float32)
    for i in range(4):
        acc += jnp.dot(hs[i][...], w1[i], preferred_element_type=jnp.float32)
    t = _lrelu(acc)
    o_ref[...] = jnp.abs(
        _lrelu(jnp.dot(t, w2[...], preferred_element_type=jnp.float32) + b2[...]))


def _head_net(h_list, w1, b1, w2, b2):
    m, n = h_list[0].shape
    w1r = w1.reshape(4, n, 64)
    return pl.pallas_call(
        _head_net_body,
        grid=(m // _BM,),
        in_specs=[pl.BlockSpec((_BM, n), lambda i: (i, 0)) for _ in range(4)] + [
            pl.BlockSpec((4, n, 64), lambda i: (0, 0, 0)),
            pl.BlockSpec((1, 64), lambda i: (0, 0)),
            pl.BlockSpec((64, 64), lambda i: (0, 0)),
            pl.BlockSpec((1, 64), lambda i: (0, 0)),
        ],
        out_specs=pl.BlockSpec((_BM, 64), lambda i: (i, 0)),
        out_shape=jax.ShapeDtypeStruct((m, 64), jnp.float32),
    )(*h_list, w1r, b1.reshape(1, 64), w2, b2.reshape(1, 64))


_NBUF = 2
_NBU = _UPAD // (_NW * _B)
_NBW = _WPAD // (_NW * _B)


@functools.lru_cache(maxsize=None)
def _sc_agg():
    mesh = plsc.VectorSubcoreMesh(core_axis_name="c", subcore_axis_name="s",
                                  num_cores=2, num_subcores=16)

    def body(cnt_ref, gu_ref, du_ref, gw_ref, dw_ref, w_ref, tab_ref, z_ref,
             out_ref, rows_v, cnt_v, gi_v, di_v, w_v, accum,
             gsem0, gsem1, ssem0, ssem1):
        gsems = (gsem0, gsem1)
        ssems = (ssem0, ssem1)
        c = lax.axis_index("c")
        s = lax.axis_index("s")
        wid = c * 16 + s

        pltpu.sync_copy(cnt_ref, cnt_v)
        lanes = lax.iota(jnp.int32, 16)
        cnt = cnt_v[...]
        nbu = jnp.sum(jnp.where(lanes == 0, cnt, 0))
        nbw = jnp.sum(jnp.where(lanes == 1, cnt, 0))

        pltpu.sync_copy(w_ref.at[wid], w_v)

        def bump(g_v, nrow, off):
            def rbody(r, cc):
                for kk in range(_B // 16):
                    g_v[r, pl.ds(kk * 16, 16)] = (
                        g_v[r, pl.ds(kk * 16, 16)] + off)
                return cc

            lax.fori_loop(0, nrow, rbody, 0)

        def run_phase(g_v, d_v, nb, weighted):
            def start_g(j, b):
                pltpu.async_copy(tab_ref.at[g_v.at[j]], rows_v.at[b],
                                 gsems[b])

            def start_s(j, b):
                pltpu.async_copy(rows_v.at[b], accum.at[d_v.at[j]],
                                 ssems[b], add=True)

            def drain(sem, b):
                pltpu.make_async_copy(tab_ref.at[g_v.at[0]], rows_v.at[b],
                                      sem).wait()

            def mult(j, b):
                def ebody(e, cc):
                    jv = jnp.zeros((16,), jnp.int32) + j
                    ev = jnp.zeros((16,), jnp.int32) + e
                    wb = plsc.load_gather(w_v, [jv, ev])
                    for kk in range(_CW // 16):
                        rows_v[b, e, pl.ds(kk * 16, 16)] = (
                            rows_v[b, e, pl.ds(kk * 16, 16)] * wb)
                    return cc

                lax.fori_loop(0, _B, ebody, 0)

            @pl.when(nb > 0)
            def _():
                start_g(0, 0)

            def pair(q, carry):
                j0 = 2 * q
                drain(gsems[0], 0)
                if weighted:
                    mult(j0, 0)

                @pl.when(q >= 1)
                def _():
                    drain(ssems[1], 1)

                start_g(j0 + 1, 1)
                start_s(j0, 0)
                drain(gsems[1], 1)
                if weighted:
                    mult(j0 + 1, 1)

                @pl.when(j0 + 2 < nb)
                def _():
                    drain(ssems[0], 0)
                    start_g(j0 + 2, 0)

                start_s(j0 + 1, 1)
                return carry

            lax.fori_loop(0, nb // 2, pair, 0)

            @pl.when(nb > 0)
            def _():
                drain(ssems[0], 0)
                drain(ssems[1], 1)

        for cf in range(_NCHUNK):
            pltpu.sync_copy(z_ref, accum.at[pl.ds(s * _RPT, _RPT)])
            pltpu.sync_copy(gu_ref.at[wid], gi_v)
            pltpu.sync_copy(du_ref.at[wid], di_v)
            if cf:
                bump(gi_v, _NBU, cf * _N_NODE)
            plsc.subcore_barrier()
            run_phase(gi_v, di_v, nbu, False)
            pltpu.sync_copy(gw_ref.at[wid], gi_v.at[pl.ds(0, _NBW)])
            pltpu.sync_copy(dw_ref.at[wid], di_v.at[pl.ds(0, _NBW)])
            if cf:
                bump(gi_v, _NBW, cf * _N_NODE)
            run_phase(gi_v, di_v, nbw, True)
            plsc.subcore_barrier()
            pltpu.sync_copy(accum.at[pl.ds(s * _RPT, _RPT)],
                            out_ref.at[c, cf].at[pl.ds(s * _RPT, _RPT)])
            plsc.subcore_barrier()

    scratch = [
        pltpu.VMEM((_NBUF, _B, _CW), jnp.float32),
        pltpu.VMEM((16,), jnp.int32),
        pltpu.VMEM((_NBU, _B), jnp.int32),
        pltpu.VMEM((_NBU, _B), jnp.int32),
        pltpu.VMEM((_NBW, _B), jnp.float32),
        pltpu.VMEM_SHARED((_ACC_ROWS, _CW), jnp.float32),
    ] + [pltpu.SemaphoreType.DMA] * 4

    return pl.kernel(
        body,
        out_type=jax.ShapeDtypeStruct((2, _NCHUNK, _ACC_ROWS, _CW),
                                      jnp.float32),
        mesh=mesh,
        compiler_params=pltpu.CompilerParams(needs_layout_passes=False),
        scratch_types=scratch,
    )


def _pad1(x, n, val):
    return jnp.concatenate(
        [x, jnp.full((n - x.shape[0],), val, x.dtype)]).reshape(_NW, -1, _B)


def kernel(node_x, net_x, edge_weight_sink_to_net, params,
           edge_index_sink_to_net, edge_index_source_to_net):
    p = params
    ew = edge_weight_sink_to_net

    h = _linear_lrelu(node_x, p['enc_W1'], p['enc_b1'])
    h = _linear_lrelu(h, p['enc_W2'], p['enc_b2'])
    hn = _linear_lrelu(net_x, p['net_W'], p['net_b'])

    src_nodes = edge_index_source_to_net[0].astype(jnp.int32)
    src_nets = edge_index_source_to_net[1].astype(jnp.int32)
    sink_all = jnp.concatenate(
        [edge_index_sink_to_net.astype(jnp.int32),
         ew[None].view(jnp.int32)], axis=0)[:, _KEEP]
    sink_nodes = sink_all[0]
    sink_nets = sink_all[1]
    kw = sink_all[2].view(jnp.float32)

    gu = _pad1(src_nodes, _UPAD, 0)
    du = _pad1(src_nets, _UPAD, _N_NET)
    gw = _pad1(sink_nodes, _WPAD, 0)
    dw = _pad1(sink_nets, _WPAD, _N_NET)
    wv = _pad1(kw, _WPAD, 0.0)
    gn = jnp.pad(_pad1(sink_nets, _WPAD, 0),
                 ((0, 0), (0, _NBU - _NBW), (0, 0)))
    dn = jnp.pad(_pad1(sink_nodes, _WPAD, _N_NET),
                 ((0, 0), (0, _NBU - _NBW), (0, 0)),
                 constant_values=_N_NET)
    zeros = jnp.zeros((_RPT, _CW), jnp.float32)
    gw_dummy = jnp.zeros((_NW, _NBW, _B), jnp.int32)
    dw_dummy = jnp.full((_NW, _NBW, _B), _N_NET, jnp.int32)
    w_dummy = jnp.zeros((_NW, _NBW, _B), jnp.float32)
    cnt_net = jnp.asarray(np.array([_NBU, _NBW] + [0] * 14, np.int32))
    cnt_node = jnp.asarray(np.array([_NBW, 0] + [0] * 14, np.int32))

    agg_fn = _sc_agg()

    ls = p['layers']
    is_net = jnp.asarray(np.array([True, False] * _NUM_LAYER))
    xs = {
        'is_net': is_net,
        'gu': jnp.stack([gu, gn] * _NUM_LAYER),
        'du': jnp.stack([du, dn] * _NUM_LAYER),
        'gw': jnp.stack([gw, gw_dummy] * _NUM_LAYER),
        'dw': jnp.stack([dw, dw_dummy] * _NUM_LAYER),
        'wv': jnp.stack([wv, w_dummy] * _NUM_LAYER),
        'cnt': jnp.stack([cnt_net, cnt_node] * _NUM_LAYER),
        'wm': jnp.stack([ls[l]['W_msg'] for l in range(_NUM_LAYER)
                         for _ in range(2)]),
        'bm': jnp.stack([ls[l]['b_msg'] for l in range(_NUM_LAYER)
                         for _ in range(2)]),
        'wt': jnp.stack([w for l in range(_NUM_LAYER)
                         for w in (ls[l]['W_net'][:_EMB],
                                   ls[l]['W_node'][:_EMB])]),
        'wb': jnp.stack([w for l in range(_NUM_LAYER)
                         for w in (ls[l]['W_net'][_EMB:],
                                   ls[l]['W_node'][_EMB:])]),
        'bb': jnp.stack([w for l in range(_NUM_LAYER)
                         for w in (ls[l]['b_net'], ls[l]['b_node'])]),
    }

    def step(carry, x):
        hc, hnc, tbl = carry
        tbl = lax.cond(x['is_net'],
                       lambda: _msg_mm(hc, x['wm'], x['bm']),
                       lambda: tbl)
        agg = agg_fn(x['cnt'], x['gu'], x['du'], x['gw'], x['dw'], x['wv'],
                     tbl.reshape(_NCHUNK * _N_NODE, _CW), zeros)

        def net_branch():
            pre_cm, hn_new = _hn_mm(hnc, agg, x['wt'], x['wb'], x['bb'])
            return hc, hn_new, pre_cm

        def node_branch():
            h_new = _h_mm(hc, agg, x['wt'], x['wb'], x['bb'])
            return h_new, hnc, tbl

        carry2 = lax.cond(x['is_net'], net_branch, node_branch)
        return carry2, (carry2[0], carry2[1])

    tbl0 = jnp.zeros((_NCHUNK, _N_NODE, _CW), jnp.float32)
    _, (hs, hns) = lax.scan(step, (h, hn, tbl0), xs)

    h_list = [h, hs[1], hs[3], hs[5]]
    hn_list = [hn, hns[0], hns[2], hns[4]]

    node_out = _head_node(h_list, p['fc1n_W'], p['fc1n_b'],
                          p['fc2n_W'], p['fc2n_b'], p['final_W'], p['final_b'])
    net_out = _head_net(hn_list, p['fc1e_W'], p['fc1e_b'],
                        p['fc2e_W'], p['fc2e_b'])
    return node_out, net_out


# speedup vs baseline: 1.4267x; 1.0243x over previous
import functools

import numpy as np
import jax
import jax.numpy as jnp
from jax import lax
from jax.experimental import pallas as pl
from jax.experimental.pallas import tpu as pltpu
from jax.experimental.pallas import tpu_sc as plsc

_NUM_LAYER = 3
_EMB = 512
_N_NODE = 10000
_N_NET = 10000
_E = 160000
_DROP_P = 0.4
_NCHUNK = 4
_CW = 128
_B = 64
_NW = 32
_ACC_ROWS = 10112
_RPT = _ACC_ROWS // 16
_BM = 400

_U32 = np.uint64(0xFFFFFFFF)


def _threefry2x32(k0, k1, x0, x1):
    x0 = np.asarray(x0, np.uint64)
    x1 = np.asarray(x1, np.uint64)
    ks = [np.uint64(k0), np.uint64(k1),
          np.uint64(k0) ^ np.uint64(k1) ^ np.uint64(0x1BD11BDA)]
    rot = ((13, 15, 26, 6), (17, 29, 16, 24))
    x0 = (x0 + ks[0]) & _U32
    x1 = (x1 + ks[1]) & _U32
    for i in range(5):
        for r in rot[i % 2]:
            x0 = (x0 + x1) & _U32
            r64 = np.uint64(r)
            x1 = ((x1 << r64 | x1 >> (np.uint64(32) - r64)) & _U32) ^ x0
        x0 = (x0 + ks[(i + 1) % 3]) & _U32
        x1 = (x1 + ks[(i + 2) % 3] + np.uint64(i + 1)) & _U32
    return x0.astype(np.uint32), x1.astype(np.uint32)


def _uniform01(seed, n):
    idx = np.arange(n, dtype=np.uint64)
    hi = (idx >> np.uint64(32)).astype(np.uint32)
    lo = (idx & _U32).astype(np.uint32)
    a, b = _threefry2x32(0, seed, hi, lo)
    bits = a ^ b
    u = ((bits >> np.uint32(9)) | np.uint32(0x3F800000)).view(np.float32)
    return np.maximum(np.float32(0.0), u - np.float32(1.0))


_MASK = _uniform01(42, _E) >= _DROP_P
_KEEP = np.nonzero(_MASK)[0].astype(np.int32)
_KN = int(_KEEP.shape[0])


def _round_up(n, m):
    return (n + m - 1) // m * m


_UPAD = _round_up(_E, 2 * _B * _NW)
_WPAD = _round_up(_KN, 2 * _B * _NW)


def _lrelu(x):
    return jnp.where(x >= 0, x, 0.1 * x)


def _lin_body(x_ref, w_ref, b_ref, o_ref):
    y = jnp.dot(x_ref[...], w_ref[...], preferred_element_type=jnp.float32)
    o_ref[...] = _lrelu(y + b_ref[...])


def _linear_lrelu(x, w, b):
    m, k = x.shape
    n = w.shape[1]
    return pl.pallas_call(
        _lin_body,
        grid=(m // _BM,),
        in_specs=[
            pl.BlockSpec((_BM, k), lambda i: (i, 0)),
            pl.BlockSpec((k, n), lambda i: (0, 0)),
            pl.BlockSpec((1, n), lambda i: (0, 0)),
        ],
        out_specs=pl.BlockSpec((_BM, n), lambda i: (i, 0)),
        out_shape=jax.ShapeDtypeStruct((m, n), jnp.float32),
    )(x, w, b.reshape(1, n))


def _msg_body(x_ref, w_ref, b_ref, o_ref):
    y = _lrelu(jnp.dot(x_ref[...], w_ref[...], preferred_element_type=jnp.float32)
               + b_ref[...])
    for c in range(_NCHUNK):
        o_ref[c] = y[:, c * _CW:(c + 1) * _CW]


def _msg_mm(x, w, b):
    m, k = x.shape
    n = w.shape[1]
    return pl.pallas_call(
        _msg_body,
        grid=(m // _BM,),
        in_specs=[
            pl.BlockSpec((_BM, k), lambda i: (i, 0)),
            pl.BlockSpec((k, n), lambda i: (0, 0)),
            pl.BlockSpec((1, n), lambda i: (0, 0)),
        ],
        out_specs=pl.BlockSpec((_NCHUNK, _BM, _CW), lambda i: (0, i, 0)),
        out_shape=jax.ShapeDtypeStruct((_NCHUNK, m, _CW), jnp.float32),
    )(x, w, b.reshape(1, n))


def _hn_body(hn_ref, agg_ref, w_ref, b_ref, opre_ref, ores_ref):
    hn = hn_ref[...]
    agg = jnp.concatenate(
        [agg_ref[0, c] + agg_ref[1, c] for c in range(_NCHUNK)], axis=1)
    w = w_ref[...]
    acc = (jnp.dot(hn, w[:_EMB], preferred_element_type=jnp.float32)
           + jnp.dot(agg, w[_EMB:], preferred_element_type=jnp.float32))
    pre = _lrelu(acc + b_ref[...])
    for c in range(_NCHUNK):
        opre_ref[c] = pre[:, c * _CW:(c + 1) * _CW]
    ores_ref[...] = pre + hn


def _hn_mm(hn, agg, w, b):
    m, n = hn.shape
    return pl.pallas_call(
        _hn_body,
        grid=(m // _BM,),
        in_specs=[
            pl.BlockSpec((_BM, n), lambda i: (i, 0)),
            pl.BlockSpec((2, _NCHUNK, _BM, _CW), lambda i: (0, 0, i, 0)),
            pl.BlockSpec((2 * n, n), lambda i: (0, 0)),
            pl.BlockSpec((1, n), lambda i: (0, 0)),
        ],
        out_specs=[
            pl.BlockSpec((_NCHUNK, _BM, _CW), lambda i: (0, i, 0)),
            pl.BlockSpec((_BM, n), lambda i: (i, 0)),
        ],
        out_shape=[
            jax.ShapeDtypeStruct((_NCHUNK, m, _CW), jnp.float32),
            jax.ShapeDtypeStruct((m, n), jnp.float32),
        ],
    )(hn, agg, w, b.reshape(1, n))


def _h_body(h_ref, agg_ref, w_ref, b_ref, o_ref):
    h = h_ref[...]
    agg = jnp.concatenate(
        [agg_ref[0, c] + agg_ref[1, c] for c in range(_NCHUNK)], axis=1)
    w = w_ref[...]
    acc = (jnp.dot(h, w[:_EMB], preferred_element_type=jnp.float32)
           + jnp.dot(agg, w[_EMB:], preferred_element_type=jnp.float32))
    o_ref[...] = _lrelu(acc + b_ref[...]) + h


def _h_mm(h, agg, w, b):
    m, n = h.shape
    return pl.pallas_call(
        _h_body,
        grid=(m // _BM,),
        in_specs=[
            pl.BlockSpec((_BM, n), lambda i: (i, 0)),
            pl.BlockSpec((2, _NCHUNK, _BM, _CW), lambda i: (0, 0, i, 0)),
            pl.BlockSpec((2 * n, n), lambda i: (0, 0)),
            pl.BlockSpec((1, n), lambda i: (0, 0)),
        ],
        out_specs=pl.BlockSpec((_BM, n), lambda i: (i, 0)),
        out_shape=jax.ShapeDtypeStruct((m, n), jnp.float32),
    )(h, agg, w, b.reshape(1, n))


def _head_node_body(h0, h1, h2, h3, w1, b1, w2, b2, wf, bf, o_ref):
    cat = jnp.concatenate([h0[...], h1[...], h2[...], h3[...]], axis=1)
    t = _lrelu(jnp.dot(cat, w1[...], preferred_element_type=jnp.float32)
               + b1[...])
    t = _lrelu(jnp.dot(t, w2[...], preferred_element_type=jnp.float32) + b2[...])
    o_ref[...] = jnp.dot(t, wf[...], preferred_element_type=jnp.float32) + bf[...]


def _head_node(h_list, w1, b1, w2, b2, wf, bf):
    m, n = h_list[0].shape
    w1r = w1
    return pl.pallas_call(
        _head_node_body,
        grid=(m // _BM,),
        in_specs=[pl.BlockSpec((_BM, n), lambda i: (i, 0)) for _ in range(4)] + [
            pl.BlockSpec((4 * n, 256), lambda i: (0, 0)),
            pl.BlockSpec((1, 256), lambda i: (0, 0)),
            pl.BlockSpec((256, 256), lambda i: (0, 0)),
            pl.BlockSpec((1, 256), lambda i: (0, 0)),
            pl.BlockSpec((256, 1), lambda i: (0, 0)),
            pl.BlockSpec((1, 1), lambda i: (0, 0)),
        ],
        out_specs=pl.BlockSpec((_BM, 1), lambda i: (i, 0)),
        out_shape=jax.ShapeDtypeStruct((m, 1), jnp.float32),
    )(*h_list, w1r, b1.reshape(1, 256), w2, b2.reshape(1, 256), wf,
      bf.reshape(1, 1))


def _head_net_body(h0, h1, h2, h3, w1, b1, w2, b2, o_ref):
    cat = jnp.concatenate([h0[...], h1[...], h2[...], h3[...]], axis=1)
    t = _lrelu(jnp.dot(cat, w1[...], preferred_element_type=jnp.float32)
               + b1[...])
    o_ref[...] = jnp.abs(
        _lrelu(jnp.dot(t, w2[...], preferred_element_type=jnp.float32) + b2[...]))


def _head_net(h_list, w1, b1, w2, b2):
    m, n = h_list[0].shape
    w1r = w1
    return pl.pallas_call(
        _head_net_body,
        grid=(m // _BM,),
        in_specs=[pl.BlockSpec((_BM, n), lambda i: (i, 0)) for _ in range(4)] + [
            pl.BlockSpec((4 * n, 64), lambda i: (0, 0)),
            pl.BlockSpec((1, 64), lambda i: (0, 0)),
            pl.BlockSpec((64, 64), lambda i: (0, 0)),
            pl.BlockSpec((1, 64), lambda i: (0, 0)),
        ],
        out_specs=pl.BlockSpec((_BM, 64), lambda i: (i, 0)),
        out_shape=jax.ShapeDtypeStruct((m, 64), jnp.float32),
    )(*h_list, w1r, b1.reshape(1, 64), w2, b2.reshape(1, 64))


_NBUF = 2
_NBU = _UPAD // (_NW * _B)
_NBW = _WPAD // (_NW * _B)


@functools.lru_cache(maxsize=None)
def _sc_agg():
    mesh = plsc.VectorSubcoreMesh(core_axis_name="c", subcore_axis_name="s",
                                  num_cores=2, num_subcores=16)

    def body(cnt_ref, gu_ref, du_ref, gw_ref, dw_ref, w_ref, tab_ref, z_ref,
             out_ref, rows_v, cnt_v, gi_v, di_v, w_v, accum,
             gsem0, gsem1, ssem0, ssem1):
        gsems = (gsem0, gsem1)
        ssems = (ssem0, ssem1)
        c = lax.axis_index("c")
        s = lax.axis_index("s")
        wid = c * 16 + s

        pltpu.sync_copy(cnt_ref, cnt_v)
        lanes = lax.iota(jnp.int32, 16)
        cnt = cnt_v[...]
        nbu = jnp.sum(jnp.where(lanes == 0, cnt, 0))
        nbw = jnp.sum(jnp.where(lanes == 1, cnt, 0))

        pltpu.sync_copy(w_ref.at[wid], w_v)

        def bump(g_v, nrow, off):
            def rbody(r, cc):
                for kk in range(_B // 16):
                    g_v[r, pl.ds(kk * 16, 16)] = (
                        g_v[r, pl.ds(kk * 16, 16)] + off)
                return cc

            lax.fori_loop(0, nrow, rbody, 0)

        def run_phase(g_v, d_v, nb, weighted):
            def start_g(j, b):
                pltpu.async_copy(tab_ref.at[g_v.at[j]], rows_v.at[b],
                                 gsems[b])

            def start_s(j, b):
                pltpu.async_copy(rows_v.at[b], accum.at[d_v.at[j]],
                                 ssems[b], add=True)

            def drain(sem, b):
                pltpu.make_async_copy(tab_ref.at[g_v.at[0]], rows_v.at[b],
                                      sem).wait()

            def mult(j, b):
                def ebody(e, cc):
                    jv = jnp.zeros((16,), jnp.int32) + j
                    ev = jnp.zeros((16,), jnp.int32) + e
                    wb = plsc.load_gather(w_v, [jv, ev])
                    for kk in range(_CW // 16):
                        rows_v[b, e, pl.ds(kk * 16, 16)] = (
                            rows_v[b, e, pl.ds(kk * 16, 16)] * wb)
                    return cc

                lax.fori_loop(0, _B, ebody, 0)

            @pl.when(nb > 0)
            def _():
                start_g(0, 0)

            def pair(q, carry):
                j0 = 2 * q
                drain(gsems[0], 0)
                if weighted:
                    mult(j0, 0)
                start_g(j0 + 1, 1)
                start_s(j0, 0)
                drain(ssems[0], 0)
                drain(gsems[1], 1)
                if weighted:
                    mult(j0 + 1, 1)

                @pl.when(j0 + 2 < nb)
                def _():
                    start_g(j0 + 2, 0)

                start_s(j0 + 1, 1)
                drain(ssems[1], 1)
                return carry

            lax.fori_loop(0, nb // 2, pair, 0)

        for cf in range(_NCHUNK):
            pltpu.sync_copy(z_ref, accum.at[pl.ds(s * _RPT, _RPT)])
            pltpu.sync_copy(gu_ref.at[wid], gi_v)
            pltpu.sync_copy(du_ref.at[wid], di_v)
            if cf:
                bump(gi_v, _NBU, cf * _N_NODE)
            plsc.subcore_barrier()
            run_phase(gi_v, di_v, nbu, False)
            pltpu.sync_copy(gw_ref.at[wid], gi_v.at[pl.ds(0, _NBW)])
            pltpu.sync_copy(dw_ref.at[wid], di_v.at[pl.ds(0, _NBW)])
            if cf:
                bump(gi_v, _NBW, cf * _N_NODE)
            run_phase(gi_v, di_v, nbw, True)
            plsc.subcore_barrier()
            pltpu.sync_copy(accum.at[pl.ds(s * _RPT, _RPT)],
                            out_ref.at[c, cf].at[pl.ds(s * _RPT, _RPT)])
            plsc.subcore_barrier()

    scratch = [
        pltpu.VMEM((_NBUF, _B, _CW), jnp.float32),
        pltpu.VMEM((16,), jnp.int32),
        pltpu.VMEM((_NBU, _B), jnp.int32),
        pltpu.VMEM((_NBU, _B), jnp.int32),
        pltpu.VMEM((_NBW, _B), jnp.float32),
        pltpu.VMEM_SHARED((_ACC_ROWS, _CW), jnp.float32),
    ] + [pltpu.SemaphoreType.DMA] * 4

    return pl.kernel(
        body,
        out_type=jax.ShapeDtypeStruct((2, _NCHUNK, _ACC_ROWS, _CW),
                                      jnp.float32),
        mesh=mesh,
        compiler_params=pltpu.CompilerParams(needs_layout_passes=False),
        scratch_types=scratch,
    )


def _pad1(x, n, val):
    return jnp.concatenate(
        [x, jnp.full((n - x.shape[0],), val, x.dtype)]).reshape(_NW, -1, _B)


def kernel(node_x, net_x, edge_weight_sink_to_net, params,
           edge_index_sink_to_net, edge_index_source_to_net):
    p = params
    ew = edge_weight_sink_to_net

    h = _linear_lrelu(node_x, p['enc_W1'], p['enc_b1'])
    h = _linear_lrelu(h, p['enc_W2'], p['enc_b2'])
    hn = _linear_lrelu(net_x, p['net_W'], p['net_b'])

    src_nodes = edge_index_source_to_net[0].astype(jnp.int32)
    src_nets = edge_index_source_to_net[1].astype(jnp.int32)
    sink_all = jnp.concatenate(
        [edge_index_sink_to_net.astype(jnp.int32),
         ew[None].view(jnp.int32)], axis=0)[:, _KEEP]
    sink_nodes = sink_all[0]
    sink_nets = sink_all[1]
    kw = sink_all[2].view(jnp.float32)

    gu = _pad1(src_nodes, _UPAD, 0)
    du = _pad1(src_nets, _UPAD, _N_NET)
    gw = _pad1(sink_nodes, _WPAD, 0)
    dw = _pad1(sink_nets, _WPAD, _N_NET)
    wv = _pad1(kw, _WPAD, 0.0)
    gn = jnp.pad(_pad1(sink_nets, _WPAD, 0),
                 ((0, 0), (0, _NBU - _NBW), (0, 0)))
    dn = jnp.pad(_pad1(sink_nodes, _WPAD, _N_NET),
                 ((0, 0), (0, _NBU - _NBW), (0, 0)),
                 constant_values=_N_NET)
    zeros = jnp.zeros((_RPT, _CW), jnp.float32)
    gw_dummy = jnp.zeros((_NW, _NBW, _B), jnp.int32)
    dw_dummy = jnp.full((_NW, _NBW, _B), _N_NET, jnp.int32)
    w_dummy = jnp.zeros((_NW, _NBW, _B), jnp.float32)
    cnt_net = jnp.asarray(np.array([_NBU, _NBW] + [0] * 14, np.int32))
    cnt_node = jnp.asarray(np.array([_NBW, 0] + [0] * 14, np.int32))

    agg_fn = _sc_agg()

    ls = p['layers']
    is_net = jnp.asarray(np.array([True, False] * _NUM_LAYER))
    xs = {
        'is_net': is_net,
        'gu': jnp.stack([gu, gn] * _NUM_LAYER),
        'du': jnp.stack([du, dn] * _NUM_LAYER),
        'gw': jnp.stack([gw, gw_dummy] * _NUM_LAYER),
        'dw': jnp.stack([dw, dw_dummy] * _NUM_LAYER),
        'wv': jnp.stack([wv, w_dummy] * _NUM_LAYER),
        'cnt': jnp.stack([cnt_net, cnt_node] * _NUM_LAYER),
        'wm': jnp.stack([ls[l]['W_msg'] for l in range(_NUM_LAYER)
                         for _ in range(2)]),
        'bm': jnp.stack([ls[l]['b_msg'] for l in range(_NUM_LAYER)
                         for _ in range(2)]),
        'wt': jnp.stack([w for l in range(_NUM_LAYER)
                         for w in (ls[l]['W_net'], ls[l]['W_node'])]),
        'bb': jnp.stack([w for l in range(_NUM_LAYER)
                         for w in (ls[l]['b_net'], ls[l]['b_node'])]),
    }

    def step(carry, x):
        hc, hnc, tbl = carry
        tbl = lax.cond(x['is_net'],
                       lambda: _msg_mm(hc, x['wm'], x['bm']),
                       lambda: tbl)
        agg = agg_fn(x['cnt'], x['gu'], x['du'], x['gw'], x['dw'], x['wv'],
                     tbl.reshape(_NCHUNK * _N_NODE, _CW), zeros)

        def net_branch():
            pre_cm, hn_new = _hn_mm(hnc, agg, x['wt'], x['bb'])
            return hc, hn_new, pre_cm

        def node_branch():
            h_new = _h_mm(hc, agg, x['wt'], x['bb'])
            return h_new, hnc, tbl

        carry2 = lax.cond(x['is_net'], net_branch, node_branch)
        return carry2, (carry2[0], carry2[1])

    tbl0 = jnp.zeros((_NCHUNK, _N_NODE, _CW), jnp.float32)
    _, (hs, hns) = lax.scan(step, (h, hn, tbl0), xs)

    h_list = [h, hs[1], hs[3], hs[5]]
    hn_list = [hn, hns[0], hns[2], hns[4]]

    node_out = _head_node(h_list, p['fc1n_W'], p['fc1n_b'],
                          p['fc2n_W'], p['fc2n_b'], p['final_W'], p['final_b'])
    net_out = _head_net(hn_list, p['fc1e_W'], p['fc1e_b'],
                        p['fc2e_W'], p['fc2e_b'])
    return node_out, net_out
